# baseline (speedup 1.0000x reference)
import jax
import jax.numpy as jnp
from jax.experimental import pallas as pl
from jax.experimental.pallas import tpu as pltpu

ALPHA = 0.7
TAU_WORD = 0.1
SUB = 8
CW = 640
NSLOT = 4


def _chunks(total, width):
    out = []
    off = 0
    while off < total:
        sz = min(width, total - off)
        out.append((off, sz))
        off += sz
    return out


def _sim_copy(sim_hbm, sim_buf, sem, tgt_ref, t, slot, step, k):
    row = tgt_ref[step * t + k]
    return pltpu.make_async_copy(
        sim_hbm.at[pl.ds(row, 1), :],
        sim_buf.at[slot, pl.ds(k, 1), :],
        sem.at[slot, k // SUB],
    )


def _wsc_kernel(
    tgt_ref, in_ref, mask_ref, tgt2_ref, sim_hbm,
    out_ref, sim_buf, pr_acc, ss_acc, ml_acc, smem_acc, sem,
):
    i = pl.program_id(0)
    n = pl.num_programs(0)
    _, t, v = in_ref.shape
    slot = jax.lax.rem(i, NSLOT)
    nxt = jax.lax.rem(i + 1, NSLOT)

    @pl.when(i == 0)
    def _prologue():
        smem_acc[0] = 0.0
        pr_acc[...] = jnp.zeros_like(pr_acc)
        ss_acc[...] = jnp.zeros_like(ss_acc)
        ml_acc[...] = jnp.zeros_like(ml_acc)
        for k in range(t):
            _sim_copy(sim_hbm, sim_buf, sem, tgt_ref, t, 0, 0, k).start()
        for k in range(t):
            _sim_copy(sim_hbm, sim_buf, sem, tgt_ref, t, 1, 1, k).start()
        for k in range(t):
            _sim_copy(sim_hbm, sim_buf, sem, tgt_ref, t, 2, 2, k).start()

    @pl.when(i + 3 < n)
    def _prefetch():
        nxt3 = jax.lax.rem(i + 3, NSLOT)
        for k in range(t):
            _sim_copy(sim_hbm, sim_buf, sem, tgt_ref, t, nxt3, i + 3, k).start()

    for off, sz in _chunks(t, SUB):
        for k in range(off, off + sz):
            _sim_copy(sim_hbm, sim_buf, sem, tgt_ref, t, slot, i, k).wait()

    for coff, cw in _chunks(v, CW):
        pr8 = jnp.zeros((SUB, cw), jnp.float32)
        ss8 = jnp.zeros((SUB, cw), jnp.float32)
        ml8 = jnp.zeros((SUB, cw), jnp.float32)
        for off, sz in _chunks(t, SUB):
            sim_t = sim_buf[slot, pl.ds(off, sz), pl.ds(coff, cw)]
            in_t = in_ref[0, pl.ds(off, sz), pl.ds(coff, cw)]
            m_t = mask_ref[0, pl.ds(off, sz), :]
            tgt_t = tgt2_ref[0, pl.ds(off, sz), :]
            smooth = jnp.exp((sim_t - 1.0) * (1.0 / TAU_WORD))
            tm = smooth * m_t
            hit = (
                coff + jax.lax.broadcasted_iota(jnp.int32, (sz, cw), 1)
            ) == tgt_t
            mlv = jnp.where(hit, in_t, 0.0) * m_t
            if sz == SUB:
                ss8 += tm
                pr8 += in_t * tm
                ml8 += mlv
            else:
                ss_acc[pl.ds(0, sz), pl.ds(coff, cw)] += tm
                pr_acc[pl.ds(0, sz), pl.ds(coff, cw)] += in_t * tm
                ml_acc[pl.ds(0, sz), pl.ds(coff, cw)] += mlv
        ss_acc[:, pl.ds(coff, cw)] += ss8
        pr_acc[:, pl.ds(coff, cw)] += pr8
        ml_acc[:, pl.ds(coff, cw)] += ml8
    smem_acc[0] += jnp.sum(mask_ref[...])

    @pl.when(i == n - 1)
    def _fin():
        ml = -jnp.sum(ml_acc[...]) / smem_acc[0]
        smooth_loss = -jnp.sum(pr_acc[...]) / jnp.sum(ss_acc[...])
        out_ref[0] = ml
        out_ref[1] = ALPHA * smooth_loss + (1.0 - ALPHA) * ml


@jax.jit
def _run(input, flat_t, mask3, tgt3, Sim_Matrix):
    b, t, v = input.shape
    grid_spec = pltpu.PrefetchScalarGridSpec(
        num_scalar_prefetch=1,
        grid=(b,),
        in_specs=[
            pl.BlockSpec((1, t, v), lambda i, tgt: (i, 0, 0)),
            pl.BlockSpec((1, t, 1), lambda i, tgt: (i, 0, 0)),
            pl.BlockSpec((1, t, 1), lambda i, tgt: (i, 0, 0)),
            pl.BlockSpec(memory_space=pltpu.HBM),
        ],
        out_specs=pl.BlockSpec(memory_space=pltpu.SMEM),
        scratch_shapes=[
            pltpu.VMEM((NSLOT, t, v), jnp.float32),
            pltpu.VMEM((SUB, v), jnp.float32),
            pltpu.VMEM((SUB, v), jnp.float32),
            pltpu.VMEM((SUB, v), jnp.float32),
            pltpu.SMEM((1,), jnp.float32),
            pltpu.SemaphoreType.DMA((NSLOT, (t + SUB - 1) // SUB)),
        ],
    )
    out = pl.pallas_call(
        _wsc_kernel,
        grid_spec=grid_spec,
        out_shape=jax.ShapeDtypeStruct((2,), jnp.float32),
    )(flat_t, input, mask3, tgt3, Sim_Matrix)
    return out[0], out[1]


def kernel(input, target, mask, Sim_Matrix):
    b, t, v = input.shape
    flat_t = target[:, :t].reshape(-1)
    mask3 = mask[:, :t].reshape(b, t, 1)
    tgt3 = target[:, :t].reshape(b, t, 1)
    return _run(input, flat_t, mask3, tgt3, Sim_Matrix)
